# Initial kernel scaffold; baseline (speedup 1.0000x reference)
import jax
import jax.numpy as jnp
from jax.experimental import pallas as pl


def kernel(pos, A, batch, edge_src, edge_dst, edge_shifts, cell, emb_table, fit_W1, fit_b1, fit_W2, fit_b2, fit_W3, fit_b3, fc_W1, fc_b1, fc_W2, fc_b2, fc_W3, fc_b3, fc_W4, fc_b4):
    raise NotImplementedError("write your pallas kernel here")


# speedup vs baseline: 3.3424x; 3.3424x over previous
import functools
import math

import jax
import jax.numpy as jnp
import numpy as np
from jax import lax
from jax.experimental import pallas as pl
from jax.experimental.pallas import tpu as pltpu
from jax.experimental.pallas import tpu_sc as plsc

NB = 16
RMAX = 5.0
U = 4
OUTM = 8
NNODES = 10000
NEDGES = 160000
NGRAPH = 16
EMBD = 16
MAXA = 10

F32 = jnp.float32
I32 = jnp.int32

_PH = jax.lax.Precision.HIGHEST
_PD = jax.lax.Precision.DEFAULT


def _dot(a, b, prec=_PH):
    return jnp.dot(a, b, precision=prec, preferred_element_type=F32)

NC = 2
NS = 16
NW = NC * NS
EPW = NEDGES // NW
CHUNK = 100
JCH = EPW // CHUNK
PAIR = 2 * CHUNK
JP = EPW // PAIR
NPAD = 10240
NROWS_PT = NPAD // NS

FEATW = 72
FPAD = 80


def _build_consts():
    r1 = np.zeros((U, U * U), np.float32)
    r2 = np.zeros((U, U * U), np.float32)
    for u in range(U):
        for v in range(U):
            r1[u, u * U + v] = 1.0
            r2[v, u * U + v] = 1.0
    r = np.zeros((U * U, U * U * 24), np.float32)
    for uv in range(U * U):
        r[uv, uv * 24:(uv + 1) * 24] = 1.0
    m = np.zeros((U * U * 24, 24), np.float32)
    for uv in range(U * U):
        for j in range(24):
            m[uv * 24 + j, j] = 1.0
    lmap = [0, 1, 1, 1, 2, 2, 2, 2, 2]
    m2 = np.zeros((24, FEATW), np.float32)
    m3 = np.zeros((9, FEATW), np.float32)
    for mm in range(9):
        for w8 in range(OUTM):
            m2[lmap[mm] * OUTM + w8, mm * OUTM + w8] = 1.0
            m3[mm, mm * OUTM + w8] = 1.0
    return r1, r2, r, m, m2, m3


_R1, _R2, _R, _M, _M2, _M3 = _build_consts()


def _prep_body(pos, a2d, bf2d, emb_pad, w1, b1, w2, b2, w3, b3, out):
    ids = lax.broadcasted_iota(I32, (NNODES, EMBD), 1)
    onehot = (a2d[...] == ids).astype(F32)
    emb = _dot(onehot, emb_pad[...])
    h = _dot(emb, w1[...], _PD) + b1[...]
    h = h / (1.0 + jnp.exp(-h))
    h = _dot(h, w2[...], _PD) + b2[...]
    h = h / (1.0 + jnp.exp(-h))
    ai = _dot(h, w3[...], _PD) + b3[...]
    out[:, 0:3] = pos[...]
    out[:, 3:7] = ai
    out[:, 7:8] = bf2d[...]
    out[:, 8:16] = jnp.zeros((NNODES, 8), F32)


def _prep_call(pos, a2d, bf2d, emb_pad, w1, b1, w2, b2, w3, b3):
    return pl.pallas_call(
        _prep_body,
        out_shape=jax.ShapeDtypeStruct((NNODES, 16), F32),
    )(pos, a2d, bf2d, emb_pad, w1, b1, w2, b2, w3, b3)


def _sc_gather_body(table, src3d, dst3d, out_s, out_d,
                    idx_s, idx_d, buf_s, buf_d, sem_s, sem_d):
    c = lax.axis_index("c")
    s = lax.axis_index("s")
    wid = s * NC + c
    pltpu.sync_copy(src3d.at[wid], idx_s)
    pltpu.sync_copy(dst3d.at[wid], idx_d)
    base = pl.multiple_of(wid * EPW, 8)

    def step(j, _):
        cp1 = pltpu.async_copy(table.at[idx_s.at[2 * j]],
                               buf_s.at[pl.ds(0, CHUNK)], sem_s)
        cp2 = pltpu.async_copy(table.at[idx_s.at[2 * j + 1]],
                               buf_s.at[pl.ds(CHUNK, CHUNK)], sem_s)
        cp3 = pltpu.async_copy(table.at[idx_d.at[2 * j]],
                               buf_d.at[pl.ds(0, CHUNK)], sem_d)
        cp4 = pltpu.async_copy(table.at[idx_d.at[2 * j + 1]],
                               buf_d.at[pl.ds(CHUNK, CHUNK)], sem_d)
        cp1.wait()
        cp2.wait()
        cp3.wait()
        cp4.wait()
        off = pl.multiple_of(base + j * PAIR, 8)
        pltpu.sync_copy(buf_s, out_s.at[pl.ds(off, PAIR)])
        pltpu.sync_copy(buf_d, out_d.at[pl.ds(off, PAIR)])
        return 0

    lax.fori_loop(0, JP, step, 0)


def _sc_gather_call(table, src3d, dst3d):
    return pl.kernel(
        _sc_gather_body,
        out_type=(
            jax.ShapeDtypeStruct((NEDGES, 16), F32),
            jax.ShapeDtypeStruct((NEDGES, 16), F32),
        ),
        mesh=plsc.VectorSubcoreMesh(
            core_axis_name="c", subcore_axis_name="s",
            num_cores=NC, num_subcores=NS),
        scratch_types=[
            pltpu.VMEM((JCH, CHUNK), I32),
            pltpu.VMEM((JCH, CHUNK), I32),
            pltpu.VMEM((PAIR, 16), F32),
            pltpu.VMEM((PAIR, 16), F32),
            pltpu.SemaphoreType.DMA,
            pltpu.SemaphoreType.DMA,
        ],
        compiler_params=pltpu.CompilerParams(use_tc_tiling_on_sc=False),
    )(table, src3d, dst3d)


_BE = 1000
_C1 = math.sqrt(3.0)
_C15 = math.sqrt(15.0)
_C5 = math.sqrt(5.0)
_ALPHA = 1.0 / math.sqrt(U * U)


def _dense_body(src, dst, shifts, cell_flat,
                w1, b1, w2, b2, w3, b3, w4p, b4p,
                r1m, r2m, rm, mm, m2m, m3m, out):
    pos_s = src[:, 0:3]
    ai_s = src[:, 3:7]
    bf = src[:, 7:8]
    pos_d = dst[:, 0:3]
    ai_d = dst[:, 3:7]

    gids = lax.broadcasted_iota(I32, (_BE, NGRAPH), 1).astype(F32)
    onehot_g = (bf == gids).astype(F32)
    crows = _dot(onehot_g, cell_flat[...])
    sh = shifts[...]
    sv = (sh[:, 0:1] * crows[:, 0:3] + sh[:, 1:2] * crows[:, 3:6]
          + sh[:, 2:3] * crows[:, 6:9])
    vec = pos_d - pos_s + sv
    r2 = jnp.sum(vec * vec, axis=1, keepdims=True)
    r = jnp.sqrt(r2)
    u = vec / jnp.maximum(r, 1e-12)
    x = u[:, 0:1]
    y = u[:, 1:2]
    z = u[:, 2:3]
    one = jnp.ones((_BE, 1), F32)
    sh9 = jnp.concatenate([
        one, _C1 * x, _C1 * y, _C1 * z,
        _C15 * x * z, _C15 * x * y,
        _C5 * (y * y - 0.5 * (x * x + z * z)),
        _C15 * y * z, (_C15 / 2.0) * (z * z - x * x),
    ], axis=1)

    xr = jnp.clip(r / RMAX, 0.0, 1.0)
    centers = (lax.broadcasted_iota(I32, (_BE, NB), 1).astype(F32)
               / float(NB - 1))
    dx = (xr - centers) * float(NB - 1)
    emb = jnp.exp(-0.5 * dx * dx) * float(NB ** 0.5)
    emb = emb * (r <= RMAX).astype(F32)

    g = _dot(emb, w1[...], _PD) + b1[...]
    g = g / (1.0 + jnp.exp(-g))
    g = _dot(g, w2[...], _PD) + b2[...]
    g = g / (1.0 + jnp.exp(-g))
    g = _dot(g, w3[...], _PD) + b3[...]
    g = g / (1.0 + jnp.exp(-g))
    w2e = _dot(g, w4p[...], _PD) + b4p[...]

    p = _dot(ai_s, r1m[...]) * _dot(ai_d, r2m[...])
    t = _dot(p, rm[...]) * w2e
    s24 = _dot(t, mm[...])
    feat = _ALPHA * _dot(sh9, m3m[...]) * _dot(s24, m2m[...])
    out[:, 0:FEATW] = feat
    out[:, FEATW:FEATW + 1] = jnp.ones((_BE, 1), F32)
    out[:, FEATW + 1:FPAD] = jnp.zeros((_BE, FPAD - FEATW - 1), F32)


def _dense_call(src_rows, dst_rows, shifts, cell_flat,
                w1, b1, w2, b2, w3, b3, w4p, b4p):
    grid = (NEDGES // _BE,)
    full = lambda a: pl.BlockSpec(a.shape, lambda i: tuple(0 for _ in a.shape))
    return pl.pallas_call(
        _dense_body,
        grid=grid,
        in_specs=[
            pl.BlockSpec((_BE, 16), lambda i: (i, 0)),
            pl.BlockSpec((_BE, 16), lambda i: (i, 0)),
            pl.BlockSpec((_BE, 3), lambda i: (i, 0)),
            full(cell_flat), full(w1), full(b1), full(w2), full(b2),
            full(w3), full(b3), full(w4p), full(b4p),
            pl.BlockSpec((U, U * U), lambda i: (0, 0)),
            pl.BlockSpec((U, U * U), lambda i: (0, 0)),
            pl.BlockSpec((U * U, 384), lambda i: (0, 0)),
            pl.BlockSpec((384, 24), lambda i: (0, 0)),
            pl.BlockSpec((24, FEATW), lambda i: (0, 0)),
            pl.BlockSpec((9, FEATW), lambda i: (0, 0)),
        ],
        out_specs=pl.BlockSpec((_BE, FPAD), lambda i: (i, 0)),
        out_shape=jax.ShapeDtypeStruct((NEDGES, FPAD), F32),
    )(src_rows, dst_rows, shifts, cell_flat, w1, b1, w2, b2, w3, b3,
      w4p, b4p, jnp.asarray(_R1), jnp.asarray(_R2), jnp.asarray(_R),
      jnp.asarray(_M), jnp.asarray(_M2), jnp.asarray(_M3))


_ZR = 128


def _sc_scatter_body(feat, dst3d, out, acc, idx, fbuf, zbuf):
    c = lax.axis_index("c")
    s = lax.axis_index("s")
    wid = s * NC + c

    def zrow(i, _):
        for k in range(FPAD // 16):
            zbuf[i, pl.ds(k * 16, 16)] = jnp.zeros((16,), F32)
        return 0

    lax.fori_loop(0, _ZR, zrow, 0)
    srow = pl.multiple_of(s * NROWS_PT, 8)

    def zcp(q, _):
        pltpu.sync_copy(zbuf, acc.at[pl.ds(srow + q * _ZR, _ZR)])
        return 0

    lax.fori_loop(0, NROWS_PT // _ZR, zcp, 0)
    plsc.subcore_barrier()

    pltpu.sync_copy(dst3d.at[wid], idx)
    base = pl.multiple_of(wid * EPW, 8)

    def step(j, _):
        off = pl.multiple_of(base + j * PAIR, 8)
        pltpu.sync_copy(feat.at[pl.ds(off, PAIR)], fbuf)
        pltpu.sync_copy(fbuf.at[pl.ds(0, CHUNK)],
                        acc.at[idx.at[2 * j]], add=True)
        pltpu.sync_copy(fbuf.at[pl.ds(CHUNK, CHUNK)],
                        acc.at[idx.at[2 * j + 1]], add=True)
        return 0

    lax.fori_loop(0, JP, step, 0)
    plsc.subcore_barrier()
    pltpu.sync_copy(acc.at[pl.ds(srow, NROWS_PT)],
                    out.at[c, pl.ds(srow, NROWS_PT)])


def _sc_scatter_call(feat, dst3d):
    return pl.kernel(
        _sc_scatter_body,
        out_type=jax.ShapeDtypeStruct((NC, NPAD, FPAD), F32),
        mesh=plsc.VectorSubcoreMesh(
            core_axis_name="c", subcore_axis_name="s",
            num_cores=NC, num_subcores=NS),
        scratch_types=[
            pltpu.VMEM_SHARED((NPAD, FPAD), F32),
            pltpu.VMEM((JCH, CHUNK), I32),
            pltpu.VMEM((PAIR, FPAD), F32),
            pltpu.VMEM((_ZR, FPAD), F32),
        ],
        compiler_params=pltpu.CompilerParams(use_tc_tiling_on_sc=False),
    )(feat, dst3d)


def _final_body(parts, out):
    tot = parts[0, 0:NNODES, :] + parts[1, 0:NNODES, :]
    cnt = jnp.maximum(tot[:, FEATW:FEATW + 1], 1.0)
    out[...] = tot[:, 0:FEATW] / cnt


def _final_call(parts):
    return pl.pallas_call(
        _final_body,
        out_shape=jax.ShapeDtypeStruct((NNODES, FEATW), F32),
    )(parts)


def kernel(pos, A, batch, edge_src, edge_dst, edge_shifts, cell, emb_table,
           fit_W1, fit_b1, fit_W2, fit_b2, fit_W3, fit_b3,
           fc_W1, fc_b1, fc_W2, fc_b2, fc_W3, fc_b3, fc_W4, fc_b4):
    a2d = A.reshape(NNODES, 1)
    bf2d = batch.astype(F32).reshape(NNODES, 1)
    emb_pad = jnp.zeros((16, EMBD), F32).at[:MAXA].set(emb_table)
    node_table = _prep_call(
        pos, a2d, bf2d, emb_pad,
        fit_W1, fit_b1.reshape(1, -1), fit_W2, fit_b2.reshape(1, -1),
        fit_W3, fit_b3.reshape(1, -1))

    src3d = edge_src.reshape(NW, JCH, CHUNK)
    dst3d = edge_dst.reshape(NW, JCH, CHUNK)
    src_rows, dst_rows = _sc_gather_call(node_table, src3d, dst3d)

    w4p = fc_W4.reshape(64, 3, U, U, OUTM).transpose(0, 2, 3, 1, 4)
    w4p = w4p.reshape(64, 3 * U * U * OUTM)
    b4p = fc_b4.reshape(3, U, U, OUTM).transpose(1, 2, 0, 3).reshape(1, -1)
    cell_flat = cell.reshape(NGRAPH, 9)
    feat = _dense_call(
        src_rows, dst_rows, edge_shifts, cell_flat,
        fc_W1, fc_b1.reshape(1, -1), fc_W2, fc_b2.reshape(1, -1),
        fc_W3, fc_b3.reshape(1, -1), w4p, b4p)

    parts = _sc_scatter_call(feat, dst3d)
    return _final_call(parts)


# speedup vs baseline: 9.1762x; 2.7454x over previous
import functools
import math

import jax
import jax.numpy as jnp
import numpy as np
from jax import lax
from jax.experimental import pallas as pl
from jax.experimental.pallas import tpu as pltpu
from jax.experimental.pallas import tpu_sc as plsc

NB = 16
RMAX = 5.0
U = 4
OUTM = 8
NNODES = 10000
NEDGES = 160000
NGRAPH = 16
EMBD = 16
MAXA = 10

F32 = jnp.float32
I32 = jnp.int32

_PH = jax.lax.Precision.HIGHEST
_PD = jax.lax.Precision.DEFAULT


def _dot(a, b, prec=_PH):
    return jnp.dot(a, b, precision=prec, preferred_element_type=F32)


def _split(a):
    ah = a.astype(jnp.bfloat16).astype(F32)
    return ah, a - ah


def _dx(a, b):
    ah, al = _split(a)
    return _dot(ah, b, _PD) + _dot(al, b, _PD)

NC = 2
NS = 16
NW = NC * NS
EPW = NEDGES // NW
CHUNK = 100
JCH = EPW // CHUNK
PAIR = 2 * CHUNK
JP = EPW // PAIR
NPAD = 10240
NROWS_PT = NPAD // NS

FEATW = 72
FPAD = 80


def _build_consts():
    r1 = np.zeros((U, U * U * 24), np.float32)
    r2 = np.zeros((U, U * U * 24), np.float32)
    for u in range(U):
        for v in range(U):
            uv = u * U + v
            r1[u, uv * 24:(uv + 1) * 24] = 1.0
            r2[v, uv * 24:(uv + 1) * 24] = 1.0
    r = None
    lmap = [0, 1, 1, 1, 2, 2, 2, 2, 2]
    m = np.zeros((U * U * 24, FEATW), np.float32)
    for uv in range(U * U):
        for mmi in range(9):
            for w8 in range(OUTM):
                m[uv * 24 + lmap[mmi] * OUTM + w8, mmi * OUTM + w8] = 1.0
    m2 = None
    m3 = np.zeros((9, FEATW), np.float32)
    for mm in range(9):
        for w8 in range(OUTM):
            m3[mm, mm * OUTM + w8] = 1.0
    rg = np.zeros((NGRAPH, NGRAPH * 3), np.float32)
    rs = np.zeros((3, NGRAPH * 3), np.float32)
    for b in range(NGRAPH):
        for i in range(3):
            rg[b, b * 3 + i] = 1.0
            rs[i, b * 3 + i] = 1.0
    ra = np.zeros((3, 9), np.float32)
    rb = np.zeros((3, 9), np.float32)
    for i in range(3):
        for j in range(3):
            ra[i, i * 3 + j] = 1.0
            rb[j, i * 3 + j] = 1.0
    w1s = np.zeros((3, 9), np.float32)
    for i in range(3):
        w1s[i, 1 + i] = 1.0
    w2s = np.zeros((9, 9), np.float32)
    w2s[2, 4] = 1.0
    w2s[1, 5] = 1.0
    w2s[4, 6] = 1.0
    w2s[0, 6] = -0.5
    w2s[8, 6] = -0.5
    w2s[5, 7] = 1.0
    w2s[8, 8] = 1.0
    w2s[0, 8] = -1.0
    b_sh = np.zeros((1, 9), np.float32)
    b_sh[0, 0] = 1.0
    c1 = math.sqrt(3.0)
    c15 = math.sqrt(15.0)
    c5 = math.sqrt(5.0)
    alpha = 1.0 / math.sqrt(U * U)
    cm = [1.0, c1, c1, c1, c15, c15, c5, c15, 0.5 * c15]
    scale72 = np.zeros((1, FEATW), np.float32)
    for mmi in range(9):
        scale72[0, mmi * OUTM:(mmi + 1) * OUTM] = alpha * cm[mmi]
    w13 = w1s @ m3
    w23 = w2s @ m3
    b72 = b_sh @ m3
    return r1, r2, m, rg, rs, ra, rb, w13, w23, b72, scale72


(_R1, _R2, _M, _RG, _RS, _RA, _RB, _W13, _W23, _B72,
 _SCALE72) = _build_consts()


def _prep_body(pos, a2d, bf2d, emb_h, emb_l, w1, b1, w2, b2, w3, b3, out):
    ids = lax.broadcasted_iota(I32, (NNODES, EMBD), 1)
    onehot = (a2d[...] == ids).astype(F32)
    emb = (_dot(onehot, emb_h[...], _PD) + _dot(onehot, emb_l[...], _PD))
    h = _dot(emb, w1[...], _PD) + b1[...]
    h = h / (1.0 + jnp.exp(-h))
    h = _dot(h, w2[...], _PD) + b2[...]
    h = h / (1.0 + jnp.exp(-h))
    ai = _dot(h, w3[...], _PD) + b3[...]
    out[:, 0:3] = pos[...]
    out[:, 3:7] = ai
    out[:, 7:8] = bf2d[...]
    out[:, 8:16] = jnp.zeros((NNODES, 8), F32)


def _prep_call(pos, a2d, bf2d, emb_pad, w1, b1, w2, b2, w3, b3):
    emb_h = emb_pad.astype(jnp.bfloat16).astype(F32)
    emb_l = emb_pad - emb_h
    return pl.pallas_call(
        _prep_body,
        out_shape=jax.ShapeDtypeStruct((NNODES, 16), F32),
    )(pos, a2d, bf2d, emb_h, emb_l, w1, b1, w2, b2, w3, b3)


def _sc_gather_body(table, src3d, dst3d, out_s, out_d,
                    idx_s, idx_d, buf_s, buf_d, sem_s, sem_d):
    c = lax.axis_index("c")
    s = lax.axis_index("s")
    wid = s * NC + c
    pltpu.sync_copy(src3d.at[wid], idx_s)
    pltpu.sync_copy(dst3d.at[wid], idx_d)
    base = pl.multiple_of(wid * EPW, 8)

    def step(j, _):
        cp1 = pltpu.async_copy(table.at[idx_s.at[2 * j]],
                               buf_s.at[pl.ds(0, CHUNK)], sem_s)
        cp2 = pltpu.async_copy(table.at[idx_s.at[2 * j + 1]],
                               buf_s.at[pl.ds(CHUNK, CHUNK)], sem_s)
        cp3 = pltpu.async_copy(table.at[idx_d.at[2 * j]],
                               buf_d.at[pl.ds(0, CHUNK)], sem_d)
        cp4 = pltpu.async_copy(table.at[idx_d.at[2 * j + 1]],
                               buf_d.at[pl.ds(CHUNK, CHUNK)], sem_d)
        cp1.wait()
        cp2.wait()
        cp3.wait()
        cp4.wait()
        off = pl.multiple_of(base + j * PAIR, 8)
        pltpu.sync_copy(buf_s, out_s.at[pl.ds(off, PAIR)])
        pltpu.sync_copy(buf_d, out_d.at[pl.ds(off, PAIR)])
        return 0

    lax.fori_loop(0, JP, step, 0)


def _sc_gather_call(table, src3d, dst3d):
    return pl.kernel(
        _sc_gather_body,
        out_type=(
            jax.ShapeDtypeStruct((NEDGES, 16), F32),
            jax.ShapeDtypeStruct((NEDGES, 16), F32),
        ),
        mesh=plsc.VectorSubcoreMesh(
            core_axis_name="c", subcore_axis_name="s",
            num_cores=NC, num_subcores=NS),
        scratch_types=[
            pltpu.VMEM((JCH, CHUNK), I32),
            pltpu.VMEM((JCH, CHUNK), I32),
            pltpu.VMEM((PAIR, 16), F32),
            pltpu.VMEM((PAIR, 16), F32),
            pltpu.SemaphoreType.DMA,
            pltpu.SemaphoreType.DMA,
        ],
        compiler_params=pltpu.CompilerParams(use_tc_tiling_on_sc=False),
    )(table, src3d, dst3d)


_BE = 2000
_C1 = math.sqrt(3.0)
_C15 = math.sqrt(15.0)
_C5 = math.sqrt(5.0)
_ALPHA = 1.0 / math.sqrt(U * U)


def _dense_body(src, dst, shifts, cell_h, cell_l,
                w1, b1, w2, b2, w3, b3, w4p, b4p,
                r1m, r2m, mm, rgm, rsm, ram, rbm,
                w13m, w23m, b72m, sc72, out):
    pos_s = src[:, 0:3]
    ai_s = src[:, 3:7]
    bf = src[:, 7:8]
    pos_d = dst[:, 0:3]
    ai_d = dst[:, 3:7]

    gids = lax.broadcasted_iota(I32, (_BE, NGRAPH), 1).astype(F32)
    onehot_g = (bf == gids).astype(F32)
    oh_exp = _dot(onehot_g, rgm[...], _PD)
    sh_h, sh_l = _split(shifts[...])
    se_h = oh_exp * _dot(sh_h, rsm[...], _PD)
    se_l = oh_exp * _dot(sh_l, rsm[...], _PD)
    sv = (_dot(se_h, cell_h[...], _PD) + _dot(se_h, cell_l[...], _PD)
          + _dot(se_l, cell_h[...], _PD))
    vec = pos_d - pos_s + sv
    r2 = jnp.sum(vec * vec, axis=1, keepdims=True)
    r = jnp.sqrt(r2)
    u3 = vec * (1.0 / jnp.maximum(r, 1e-12))
    u3h, u3l = _split(u3)
    uu = ((_dot(u3h, ram[...], _PD) + _dot(u3l, ram[...], _PD))
          * (_dot(u3h, rbm[...], _PD) + _dot(u3l, rbm[...], _PD)))
    sh_exp = (b72m[...]
              + _dot(u3h, w13m[...], _PD) + _dot(u3l, w13m[...], _PD)
              + _dx(uu, w23m[...]))

    xr = jnp.minimum(r * (1.0 / RMAX), 1.0)
    centers = (lax.broadcasted_iota(I32, (_BE, NB), 1).astype(F32)
               / float(NB - 1))
    dx = (xr - centers) * float(NB - 1)
    emb = jnp.exp(-0.5 * dx * dx)
    emb = emb * jnp.where(r <= RMAX, float(NB ** 0.5), 0.0)

    g = _dot(emb, w1[...], _PD) + b1[...]
    g = g / (1.0 + jnp.exp(-g))
    g = _dot(g, w2[...], _PD) + b2[...]
    g = g / (1.0 + jnp.exp(-g))
    g = _dot(g, w3[...], _PD) + b3[...]
    g = g / (1.0 + jnp.exp(-g))
    w2e = _dot(g, w4p[...], _PD) + b4p[...]

    p_exp = _dx(ai_s, r1m[...]) * _dx(ai_d, r2m[...])
    t = p_exp * w2e
    s72 = _dx(t, mm[...])
    feat = sh_exp * s72 * sc72[...]
    out[:, 0:FEATW] = feat
    out[:, FEATW:FEATW + 1] = jnp.ones((_BE, 1), F32)
    out[:, FEATW + 1:FPAD] = jnp.zeros((_BE, FPAD - FEATW - 1), F32)


def _dense_call(src_rows, dst_rows, shifts, cell_cm,
                w1, b1, w2, b2, w3, b3, w4p, b4p):
    grid = (NEDGES // _BE,)
    full = lambda a: pl.BlockSpec(a.shape, lambda i: tuple(0 for _ in a.shape))
    cell_h = cell_cm.astype(jnp.bfloat16).astype(F32)
    cell_l = cell_cm - cell_h
    consts = [jnp.asarray(x) for x in
              (_R1, _R2, _M, _RG, _RS, _RA, _RB,
               _W13, _W23, _B72, _SCALE72)]
    return pl.pallas_call(
        _dense_body,
        grid=grid,
        in_specs=[
            pl.BlockSpec((_BE, 16), lambda i: (i, 0)),
            pl.BlockSpec((_BE, 16), lambda i: (i, 0)),
            pl.BlockSpec((_BE, 3), lambda i: (i, 0)),
            full(cell_h), full(cell_l), full(w1), full(b1), full(w2),
            full(b2), full(w3), full(b3), full(w4p), full(b4p),
        ] + [full(x) for x in consts],
        out_specs=pl.BlockSpec((_BE, FPAD), lambda i: (i, 0)),
        out_shape=jax.ShapeDtypeStruct((NEDGES, FPAD), F32),
    )(src_rows, dst_rows, shifts, cell_h, cell_l, w1, b1, w2, b2, w3, b3,
      w4p, b4p, *consts)


_ZR = 128


def _sc_scatter_body(feat, dst3d, out, acc, idx, fbuf, zbuf):
    c = lax.axis_index("c")
    s = lax.axis_index("s")
    wid = s * NC + c

    def zrow(i, _):
        for k in range(FPAD // 16):
            zbuf[i, pl.ds(k * 16, 16)] = jnp.zeros((16,), F32)
        return 0

    lax.fori_loop(0, _ZR, zrow, 0)
    srow = pl.multiple_of(s * NROWS_PT, 8)

    def zcp(q, _):
        pltpu.sync_copy(zbuf, acc.at[pl.ds(srow + q * _ZR, _ZR)])
        return 0

    lax.fori_loop(0, NROWS_PT // _ZR, zcp, 0)
    plsc.subcore_barrier()

    pltpu.sync_copy(dst3d.at[wid], idx)
    base = pl.multiple_of(wid * EPW, 8)

    def step(j, _):
        off = pl.multiple_of(base + j * PAIR, 8)
        pltpu.sync_copy(feat.at[pl.ds(off, PAIR)], fbuf)
        pltpu.sync_copy(fbuf.at[pl.ds(0, CHUNK)],
                        acc.at[idx.at[2 * j]], add=True)
        pltpu.sync_copy(fbuf.at[pl.ds(CHUNK, CHUNK)],
                        acc.at[idx.at[2 * j + 1]], add=True)
        return 0

    lax.fori_loop(0, JP, step, 0)
    plsc.subcore_barrier()
    pltpu.sync_copy(acc.at[pl.ds(srow, NROWS_PT)],
                    out.at[c, pl.ds(srow, NROWS_PT)])


def _sc_scatter_call(feat, dst3d):
    return pl.kernel(
        _sc_scatter_body,
        out_type=jax.ShapeDtypeStruct((NC, NPAD, FPAD), F32),
        mesh=plsc.VectorSubcoreMesh(
            core_axis_name="c", subcore_axis_name="s",
            num_cores=NC, num_subcores=NS),
        scratch_types=[
            pltpu.VMEM_SHARED((NPAD, FPAD), F32),
            pltpu.VMEM((JCH, CHUNK), I32),
            pltpu.VMEM((PAIR, FPAD), F32),
            pltpu.VMEM((_ZR, FPAD), F32),
        ],
        compiler_params=pltpu.CompilerParams(use_tc_tiling_on_sc=False),
    )(feat, dst3d)


def _final_body(parts, out):
    tot = parts[0, 0:NNODES, :] + parts[1, 0:NNODES, :]
    cnt = jnp.maximum(tot[:, FEATW:FEATW + 1], 1.0)
    out[...] = tot[:, 0:FEATW] / cnt


def _final_call(parts):
    return pl.pallas_call(
        _final_body,
        out_shape=jax.ShapeDtypeStruct((NNODES, FEATW), F32),
    )(parts)


def kernel(pos, A, batch, edge_src, edge_dst, edge_shifts, cell, emb_table,
           fit_W1, fit_b1, fit_W2, fit_b2, fit_W3, fit_b3,
           fc_W1, fc_b1, fc_W2, fc_b2, fc_W3, fc_b3, fc_W4, fc_b4):
    a2d = A.reshape(NNODES, 1)
    bf2d = batch.astype(F32).reshape(NNODES, 1)
    emb_pad = jnp.zeros((16, EMBD), F32).at[:MAXA].set(emb_table)
    node_table = _prep_call(
        pos, a2d, bf2d, emb_pad,
        fit_W1, fit_b1.reshape(1, -1), fit_W2, fit_b2.reshape(1, -1),
        fit_W3, fit_b3.reshape(1, -1))

    src3d = edge_src.reshape(NW, JCH, CHUNK)
    dst3d = edge_dst.reshape(NW, JCH, CHUNK)
    src_rows, dst_rows = _sc_gather_call(node_table, src3d, dst3d)

    w4p = fc_W4.reshape(64, 3, U, U, OUTM).transpose(0, 2, 3, 1, 4)
    w4p = w4p.reshape(64, 3 * U * U * OUTM)
    b4p = fc_b4.reshape(3, U, U, OUTM).transpose(1, 2, 0, 3).reshape(1, -1)
    cell_cm = cell.reshape(NGRAPH * 3, 3)
    feat = _dense_call(
        src_rows, dst_rows, edge_shifts, cell_cm,
        fc_W1, fc_b1.reshape(1, -1), fc_W2, fc_b2.reshape(1, -1),
        fc_W3, fc_b3.reshape(1, -1), w4p, b4p)

    parts = _sc_scatter_call(feat, dst3d)
    return _final_call(parts)


# speedup vs baseline: 9.3883x; 1.0231x over previous
import functools
import math

import jax
import jax.numpy as jnp
import numpy as np
from jax import lax
from jax.experimental import pallas as pl
from jax.experimental.pallas import tpu as pltpu
from jax.experimental.pallas import tpu_sc as plsc

NB = 16
RMAX = 5.0
U = 4
OUTM = 8
NNODES = 10000
NEDGES = 160000
NGRAPH = 16
EMBD = 16
MAXA = 10

F32 = jnp.float32
I32 = jnp.int32

_PH = jax.lax.Precision.HIGHEST
_PD = jax.lax.Precision.DEFAULT


def _dot(a, b, prec=_PH):
    return jnp.dot(a, b, precision=prec, preferred_element_type=F32)


def _split(a):
    ah = a.astype(jnp.bfloat16).astype(F32)
    return ah, a - ah


def _dx(a, b):
    ah, al = _split(a)
    return _dot(ah, b, _PD) + _dot(al, b, _PD)

NC = 2
NS = 16
NW = NC * NS
EPW = NEDGES // NW
CHUNK = 100
JCH = EPW // CHUNK
PAIR = 2 * CHUNK
JP = EPW // PAIR
MC = 1000
MCQ = MC // CHUNK
JG = EPW // MC
NPAD = 10240
NROWS_PT = NPAD // NS

FEATW = 72
FPAD = 80


def _build_consts():
    r1 = np.zeros((U, U * U * 24), np.float32)
    r2 = np.zeros((U, U * U * 24), np.float32)
    for u in range(U):
        for v in range(U):
            uv = u * U + v
            r1[u, uv * 24:(uv + 1) * 24] = 1.0
            r2[v, uv * 24:(uv + 1) * 24] = 1.0
    r = None
    lmap = [0, 1, 1, 1, 2, 2, 2, 2, 2]
    m = np.zeros((U * U * 24, FEATW), np.float32)
    for uv in range(U * U):
        for mmi in range(9):
            for w8 in range(OUTM):
                m[uv * 24 + lmap[mmi] * OUTM + w8, mmi * OUTM + w8] = 1.0
    m2 = None
    m3 = np.zeros((9, FEATW), np.float32)
    for mm in range(9):
        for w8 in range(OUTM):
            m3[mm, mm * OUTM + w8] = 1.0
    rg = np.zeros((NGRAPH, NGRAPH * 3), np.float32)
    rs = np.zeros((3, NGRAPH * 3), np.float32)
    for b in range(NGRAPH):
        for i in range(3):
            rg[b, b * 3 + i] = 1.0
            rs[i, b * 3 + i] = 1.0
    ra = np.zeros((3, 9), np.float32)
    rb = np.zeros((3, 9), np.float32)
    for i in range(3):
        for j in range(3):
            ra[i, i * 3 + j] = 1.0
            rb[j, i * 3 + j] = 1.0
    w1s = np.zeros((3, 9), np.float32)
    for i in range(3):
        w1s[i, 1 + i] = 1.0
    w2s = np.zeros((9, 9), np.float32)
    w2s[2, 4] = 1.0
    w2s[1, 5] = 1.0
    w2s[4, 6] = 1.0
    w2s[0, 6] = -0.5
    w2s[8, 6] = -0.5
    w2s[5, 7] = 1.0
    w2s[8, 8] = 1.0
    w2s[0, 8] = -1.0
    b_sh = np.zeros((1, 9), np.float32)
    b_sh[0, 0] = 1.0
    c1 = math.sqrt(3.0)
    c15 = math.sqrt(15.0)
    c5 = math.sqrt(5.0)
    alpha = 1.0 / math.sqrt(U * U)
    cm = [1.0, c1, c1, c1, c15, c15, c5, c15, 0.5 * c15]
    scale72 = np.zeros((1, FEATW), np.float32)
    for mmi in range(9):
        scale72[0, mmi * OUTM:(mmi + 1) * OUTM] = alpha * cm[mmi]
    w13 = w1s @ m3
    w23 = w2s @ m3
    b72 = b_sh @ m3
    return r1, r2, m, rg, rs, ra, rb, w13, w23, b72, scale72


(_R1, _R2, _M, _RG, _RS, _RA, _RB, _W13, _W23, _B72,
 _SCALE72) = _build_consts()


def _prep_body(pos, a2d, bf2d, emb_h, emb_l, w1, b1, w2, b2, w3, b3, out):
    ids = lax.broadcasted_iota(I32, (NNODES, EMBD), 1)
    onehot = (a2d[...] == ids).astype(F32)
    emb = (_dot(onehot, emb_h[...], _PD) + _dot(onehot, emb_l[...], _PD))
    h = _dot(emb, w1[...], _PD) + b1[...]
    h = h / (1.0 + jnp.exp(-h))
    h = _dot(h, w2[...], _PD) + b2[...]
    h = h / (1.0 + jnp.exp(-h))
    ai = _dot(h, w3[...], _PD) + b3[...]
    out[:, 0:3] = pos[...]
    out[:, 3:7] = ai
    out[:, 7:8] = bf2d[...]
    out[:, 8:16] = jnp.zeros((NNODES, 8), F32)


def _prep_call(pos, a2d, bf2d, emb_pad, w1, b1, w2, b2, w3, b3):
    emb_h = emb_pad.astype(jnp.bfloat16).astype(F32)
    emb_l = emb_pad - emb_h
    return pl.pallas_call(
        _prep_body,
        out_shape=jax.ShapeDtypeStruct((NNODES, 16), F32),
    )(pos, a2d, bf2d, emb_h, emb_l, w1, b1, w2, b2, w3, b3)


def _sc_gather_body(table, src3d, dst3d, out_s, out_d,
                    idx_s, idx_d, bsa, bda, bsb, bdb, sem_s, sem_d):
    c = lax.axis_index("c")
    s = lax.axis_index("s")
    wid = s * NC + c
    pltpu.sync_copy(src3d.at[wid], idx_s)
    pltpu.sync_copy(dst3d.at[wid], idx_d)
    base = pl.multiple_of(wid * EPW, 8)

    def fire(k, bs, bd):
        for q in range(MCQ):
            pltpu.async_copy(table.at[idx_s.at[k * MCQ + q]],
                             bs.at[pl.ds(q * CHUNK, CHUNK)], sem_s)
            pltpu.async_copy(table.at[idx_d.at[k * MCQ + q]],
                             bd.at[pl.ds(q * CHUNK, CHUNK)], sem_d)

    def drain(k, bs, bd):
        pltpu.make_async_copy(table.at[pl.ds(0, MC)], bs, sem_s).wait()
        pltpu.make_async_copy(table.at[pl.ds(0, MC)], bd, sem_d).wait()
        off = pl.multiple_of(base + k * MC, 8)
        pltpu.sync_copy(bs, out_s.at[pl.ds(off, MC)])
        pltpu.sync_copy(bd, out_d.at[pl.ds(off, MC)])

    fire(0, bsa, bda)

    def body(jj, _):
        k = 2 * jj
        fire(k + 1, bsb, bdb)
        drain(k, bsa, bda)
        fire(k + 2, bsa, bda)
        drain(k + 1, bsb, bdb)
        return 0

    lax.fori_loop(0, (JG - 1) // 2, body, 0)
    drain(JG - 1, bsa, bda)


def _sc_gather_call(table, src3d, dst3d):
    return pl.kernel(
        _sc_gather_body,
        out_type=(
            jax.ShapeDtypeStruct((NEDGES, 16), F32),
            jax.ShapeDtypeStruct((NEDGES, 16), F32),
        ),
        mesh=plsc.VectorSubcoreMesh(
            core_axis_name="c", subcore_axis_name="s",
            num_cores=NC, num_subcores=NS),
        scratch_types=[
            pltpu.VMEM((JCH, CHUNK), I32),
            pltpu.VMEM((JCH, CHUNK), I32),
            pltpu.VMEM((MC, 16), F32),
            pltpu.VMEM((MC, 16), F32),
            pltpu.VMEM((MC, 16), F32),
            pltpu.VMEM((MC, 16), F32),
            pltpu.SemaphoreType.DMA,
            pltpu.SemaphoreType.DMA,
        ],
        compiler_params=pltpu.CompilerParams(use_tc_tiling_on_sc=False),
    )(table, src3d, dst3d)


_BE = 2000
_C1 = math.sqrt(3.0)
_C15 = math.sqrt(15.0)
_C5 = math.sqrt(5.0)
_ALPHA = 1.0 / math.sqrt(U * U)


def _dense_body(src, dst, shifts, cell_h, cell_l,
                w1, b1, w2, b2, w3, b3, w4p, b4p,
                r1m, r2m, mm, rgm, rsm, ram, rbm,
                w13m, w23m, b72m, sc72, out):
    pos_s = src[:, 0:3]
    ai_s = src[:, 3:7]
    bf = src[:, 7:8]
    pos_d = dst[:, 0:3]
    ai_d = dst[:, 3:7]

    gids = lax.broadcasted_iota(I32, (_BE, NGRAPH), 1).astype(F32)
    onehot_g = (bf == gids).astype(F32)
    oh_exp = _dot(onehot_g, rgm[...], _PD)
    sh_h, sh_l = _split(shifts[...])
    se_h = oh_exp * _dot(sh_h, rsm[...], _PD)
    se_l = oh_exp * _dot(sh_l, rsm[...], _PD)
    sv = (_dot(se_h, cell_h[...], _PD) + _dot(se_h, cell_l[...], _PD)
          + _dot(se_l, cell_h[...], _PD))
    vec = pos_d - pos_s + sv
    r2 = jnp.sum(vec * vec, axis=1, keepdims=True)
    r = jnp.sqrt(r2)
    u3 = vec * (1.0 / jnp.maximum(r, 1e-12))
    u3h, u3l = _split(u3)
    uu = ((_dot(u3h, ram[...], _PD) + _dot(u3l, ram[...], _PD))
          * (_dot(u3h, rbm[...], _PD) + _dot(u3l, rbm[...], _PD)))
    sh_exp = (b72m[...]
              + _dot(u3h, w13m[...], _PD) + _dot(u3l, w13m[...], _PD)
              + _dx(uu, w23m[...]))

    xr = jnp.minimum(r * (1.0 / RMAX), 1.0)
    centers = (lax.broadcasted_iota(I32, (_BE, NB), 1).astype(F32)
               / float(NB - 1))
    dx = (xr - centers) * float(NB - 1)
    emb = jnp.exp(-0.5 * dx * dx)
    emb = emb * jnp.where(r <= RMAX, float(NB ** 0.5), 0.0)

    g = _dot(emb, w1[...], _PD) + b1[...]
    g = g / (1.0 + jnp.exp(-g))
    g = _dot(g, w2[...], _PD) + b2[...]
    g = g / (1.0 + jnp.exp(-g))
    g = _dot(g, w3[...], _PD) + b3[...]
    g = g / (1.0 + jnp.exp(-g))
    w2e = _dot(g, w4p[...], _PD) + b4p[...]

    p_exp = _dx(ai_s, r1m[...]) * _dx(ai_d, r2m[...])
    t = p_exp * w2e
    s72 = _dx(t, mm[...])
    feat = sh_exp * s72 * sc72[...]
    out[:, 0:FEATW] = feat
    out[:, FEATW:FEATW + 1] = jnp.ones((_BE, 1), F32)
    out[:, FEATW + 1:FPAD] = jnp.zeros((_BE, FPAD - FEATW - 1), F32)


def _dense_call(src_rows, dst_rows, shifts, cell_cm,
                w1, b1, w2, b2, w3, b3, w4p, b4p):
    grid = (NEDGES // _BE,)
    full = lambda a: pl.BlockSpec(a.shape, lambda i: tuple(0 for _ in a.shape))
    cell_h = cell_cm.astype(jnp.bfloat16).astype(F32)
    cell_l = cell_cm - cell_h
    consts = [jnp.asarray(x) for x in
              (_R1, _R2, _M, _RG, _RS, _RA, _RB,
               _W13, _W23, _B72, _SCALE72)]
    return pl.pallas_call(
        _dense_body,
        grid=grid,
        in_specs=[
            pl.BlockSpec((_BE, 16), lambda i: (i, 0)),
            pl.BlockSpec((_BE, 16), lambda i: (i, 0)),
            pl.BlockSpec((_BE, 3), lambda i: (i, 0)),
            full(cell_h), full(cell_l), full(w1), full(b1), full(w2),
            full(b2), full(w3), full(b3), full(w4p), full(b4p),
        ] + [full(x) for x in consts],
        out_specs=pl.BlockSpec((_BE, FPAD), lambda i: (i, 0)),
        out_shape=jax.ShapeDtypeStruct((NEDGES, FPAD), F32),
    )(src_rows, dst_rows, shifts, cell_h, cell_l, w1, b1, w2, b2, w3, b3,
      w4p, b4p, *consts)


_ZR = 128


def _sc_scatter_body(feat, dst3d, out, acc, idx, fba, fbb, zbuf, sem):
    c = lax.axis_index("c")
    s = lax.axis_index("s")
    wid = s * NC + c

    def zrow(i, _):
        for k in range(FPAD // 16):
            zbuf[i, pl.ds(k * 16, 16)] = jnp.zeros((16,), F32)
        return 0

    lax.fori_loop(0, _ZR, zrow, 0)
    srow = pl.multiple_of(s * NROWS_PT, 8)

    def zcp(q, _):
        pltpu.sync_copy(zbuf, acc.at[pl.ds(srow + q * _ZR, _ZR)])
        return 0

    lax.fori_loop(0, NROWS_PT // _ZR, zcp, 0)
    plsc.subcore_barrier()

    pltpu.sync_copy(dst3d.at[wid], idx)
    base = pl.multiple_of(wid * EPW, 8)

    def fire(j, fb):
        off = pl.multiple_of(base + j * PAIR, 8)
        pltpu.async_copy(feat.at[pl.ds(off, PAIR)], fb, sem)

    def drain(j, fb):
        off = pl.multiple_of(base + j * PAIR, 8)
        pltpu.make_async_copy(feat.at[pl.ds(off, PAIR)], fb, sem).wait()
        pltpu.sync_copy(fb.at[pl.ds(0, CHUNK)],
                        acc.at[idx.at[2 * j]], add=True)
        pltpu.sync_copy(fb.at[pl.ds(CHUNK, CHUNK)],
                        acc.at[idx.at[2 * j + 1]], add=True)

    fire(0, fba)

    def body(jj, _):
        j = 2 * jj
        fire(j + 1, fbb)
        drain(j, fba)
        fire(j + 2, fba)
        drain(j + 1, fbb)
        return 0

    lax.fori_loop(0, (JP - 1) // 2, body, 0)
    drain(JP - 1, fba)
    plsc.subcore_barrier()
    pltpu.sync_copy(acc.at[pl.ds(srow, NROWS_PT)],
                    out.at[c, pl.ds(srow, NROWS_PT)])


def _sc_scatter_call(feat, dst3d):
    return pl.kernel(
        _sc_scatter_body,
        out_type=jax.ShapeDtypeStruct((NC, NPAD, FPAD), F32),
        mesh=plsc.VectorSubcoreMesh(
            core_axis_name="c", subcore_axis_name="s",
            num_cores=NC, num_subcores=NS),
        scratch_types=[
            pltpu.VMEM_SHARED((NPAD, FPAD), F32),
            pltpu.VMEM((JCH, CHUNK), I32),
            pltpu.VMEM((PAIR, FPAD), F32),
            pltpu.VMEM((PAIR, FPAD), F32),
            pltpu.VMEM((_ZR, FPAD), F32),
            pltpu.SemaphoreType.DMA,
        ],
        compiler_params=pltpu.CompilerParams(use_tc_tiling_on_sc=False),
    )(feat, dst3d)


def _final_body(parts, out):
    tot = parts[0, 0:NNODES, :] + parts[1, 0:NNODES, :]
    cnt = jnp.maximum(tot[:, FEATW:FEATW + 1], 1.0)
    out[...] = tot[:, 0:FEATW] / cnt


def _final_call(parts):
    return pl.pallas_call(
        _final_body,
        out_shape=jax.ShapeDtypeStruct((NNODES, FEATW), F32),
    )(parts)


def kernel(pos, A, batch, edge_src, edge_dst, edge_shifts, cell, emb_table,
           fit_W1, fit_b1, fit_W2, fit_b2, fit_W3, fit_b3,
           fc_W1, fc_b1, fc_W2, fc_b2, fc_W3, fc_b3, fc_W4, fc_b4):
    a2d = A.reshape(NNODES, 1)
    bf2d = batch.astype(F32).reshape(NNODES, 1)
    emb_pad = jnp.zeros((16, EMBD), F32).at[:MAXA].set(emb_table)
    node_table = _prep_call(
        pos, a2d, bf2d, emb_pad,
        fit_W1, fit_b1.reshape(1, -1), fit_W2, fit_b2.reshape(1, -1),
        fit_W3, fit_b3.reshape(1, -1))

    src3d = edge_src.reshape(NW, JCH, CHUNK)
    dst3d = edge_dst.reshape(NW, JCH, CHUNK)
    src_rows, dst_rows = _sc_gather_call(node_table, src3d, dst3d)

    w4p = fc_W4.reshape(64, 3, U, U, OUTM).transpose(0, 2, 3, 1, 4)
    w4p = w4p.reshape(64, 3 * U * U * OUTM)
    b4p = fc_b4.reshape(3, U, U, OUTM).transpose(1, 2, 0, 3).reshape(1, -1)
    cell_cm = cell.reshape(NGRAPH * 3, 3)
    feat = _dense_call(
        src_rows, dst_rows, edge_shifts, cell_cm,
        fc_W1, fc_b1.reshape(1, -1), fc_W2, fc_b2.reshape(1, -1),
        fc_W3, fc_b3.reshape(1, -1), w4p, b4p)

    parts = _sc_scatter_call(feat, dst3d)
    return _final_call(parts)


# speedup vs baseline: 9.5215x; 1.0142x over previous
import functools
import math

import jax
import jax.numpy as jnp
import numpy as np
from jax import lax
from jax.experimental import pallas as pl
from jax.experimental.pallas import tpu as pltpu
from jax.experimental.pallas import tpu_sc as plsc

NB = 16
RMAX = 5.0
U = 4
OUTM = 8
NNODES = 10000
NEDGES = 160000
NGRAPH = 16
EMBD = 16
MAXA = 10

F32 = jnp.float32
I32 = jnp.int32

_PH = jax.lax.Precision.HIGHEST
_PD = jax.lax.Precision.DEFAULT


def _dot(a, b, prec=_PH):
    return jnp.dot(a, b, precision=prec, preferred_element_type=F32)


def _split(a):
    ah = a.astype(jnp.bfloat16).astype(F32)
    return ah, a - ah


def _dx(a, b):
    ah, al = _split(a)
    return _dot(ah, b, _PD) + _dot(al, b, _PD)

NC = 2
NS = 16
NW = NC * NS
EPW = NEDGES // NW
CHUNK = 100
JCH = EPW // CHUNK
PAIR = 2 * CHUNK
JP = EPW // PAIR
MC = 1000
MCQ = MC // CHUNK
JG = EPW // MC
NPAD = 10240
NROWS_PT = NPAD // NS

FEATW = 72
FPAD = 80


def _build_consts():
    r1 = np.zeros((U, U * U * 24), np.float32)
    r2 = np.zeros((U, U * U * 24), np.float32)
    for u in range(U):
        for v in range(U):
            uv = u * U + v
            r1[u, uv * 24:(uv + 1) * 24] = 1.0
            r2[v, uv * 24:(uv + 1) * 24] = 1.0
    r = None
    lmap = [0, 1, 1, 1, 2, 2, 2, 2, 2]
    m = np.zeros((U * U * 24, FEATW), np.float32)
    for uv in range(U * U):
        for mmi in range(9):
            for w8 in range(OUTM):
                m[uv * 24 + lmap[mmi] * OUTM + w8, mmi * OUTM + w8] = 1.0
    m2 = None
    m3 = np.zeros((9, FEATW), np.float32)
    for mm in range(9):
        for w8 in range(OUTM):
            m3[mm, mm * OUTM + w8] = 1.0
    rg = np.zeros((NGRAPH, NGRAPH * 3), np.float32)
    rs = np.zeros((3, NGRAPH * 3), np.float32)
    for b in range(NGRAPH):
        for i in range(3):
            rg[b, b * 3 + i] = 1.0
            rs[i, b * 3 + i] = 1.0
    ra = np.zeros((3, 9), np.float32)
    rb = np.zeros((3, 9), np.float32)
    for i in range(3):
        for j in range(3):
            ra[i, i * 3 + j] = 1.0
            rb[j, i * 3 + j] = 1.0
    w1s = np.zeros((3, 9), np.float32)
    for i in range(3):
        w1s[i, 1 + i] = 1.0
    w2s = np.zeros((9, 9), np.float32)
    w2s[2, 4] = 1.0
    w2s[1, 5] = 1.0
    w2s[4, 6] = 1.0
    w2s[0, 6] = -0.5
    w2s[8, 6] = -0.5
    w2s[5, 7] = 1.0
    w2s[8, 8] = 1.0
    w2s[0, 8] = -1.0
    b_sh = np.zeros((1, 9), np.float32)
    b_sh[0, 0] = 1.0
    c1 = math.sqrt(3.0)
    c15 = math.sqrt(15.0)
    c5 = math.sqrt(5.0)
    alpha = 1.0 / math.sqrt(U * U)
    cm = [1.0, c1, c1, c1, c15, c15, c5, c15, 0.5 * c15]
    scale72 = np.zeros((1, FEATW), np.float32)
    for mmi in range(9):
        scale72[0, mmi * OUTM:(mmi + 1) * OUTM] = alpha * cm[mmi]
    w13 = w1s @ m3
    w23 = w2s @ m3
    b72 = b_sh @ m3
    return r1, r2, m, rg, rs, ra, rb, w13, w23, b72, scale72


(_R1, _R2, _M, _RG, _RS, _RA, _RB, _W13, _W23, _B72,
 _SCALE72) = _build_consts()


def _prep_body(pos, a2d, bf2d, emb_h, emb_l, w1, b1, w2, b2, w3, b3, out):
    ids = lax.broadcasted_iota(I32, (NNODES, EMBD), 1)
    onehot = (a2d[...] == ids).astype(F32)
    emb = (_dot(onehot, emb_h[...], _PD) + _dot(onehot, emb_l[...], _PD))
    h = _dot(emb, w1[...], _PD) + b1[...]
    h = h / (1.0 + jnp.exp(-h))
    h = _dot(h, w2[...], _PD) + b2[...]
    h = h / (1.0 + jnp.exp(-h))
    ai = _dot(h, w3[...], _PD) + b3[...]
    out[:, 0:3] = pos[...]
    out[:, 3:7] = ai
    out[:, 7:8] = bf2d[...]
    out[:, 8:16] = jnp.zeros((NNODES, 8), F32)


def _prep_call(pos, a2d, bf2d, emb_pad, w1, b1, w2, b2, w3, b3):
    emb_h = emb_pad.astype(jnp.bfloat16).astype(F32)
    emb_l = emb_pad - emb_h
    return pl.pallas_call(
        _prep_body,
        out_shape=jax.ShapeDtypeStruct((NNODES, 16), F32),
    )(pos, a2d, bf2d, emb_h, emb_l, w1, b1, w2, b2, w3, b3)


def _sc_gather_body(table, src3d, dst3d, out_s, out_d,
                    idx_s, idx_d, bsa, bda, bsb, bdb, sem_s, sem_d):
    c = lax.axis_index("c")
    s = lax.axis_index("s")
    wid = s * NC + c
    pltpu.sync_copy(src3d.at[wid], idx_s)
    pltpu.sync_copy(dst3d.at[wid], idx_d)
    base = pl.multiple_of(wid * EPW, 8)

    def fire(k, bs, bd):
        for q in range(MCQ):
            pltpu.async_copy(table.at[idx_s.at[k * MCQ + q]],
                             bs.at[pl.ds(q * CHUNK, CHUNK)], sem_s)
            pltpu.async_copy(table.at[idx_d.at[k * MCQ + q]],
                             bd.at[pl.ds(q * CHUNK, CHUNK)], sem_d)

    def drain(k, bs, bd):
        pltpu.make_async_copy(table.at[pl.ds(0, MC)], bs, sem_s).wait()
        pltpu.make_async_copy(table.at[pl.ds(0, MC)], bd, sem_d).wait()
        off = pl.multiple_of(base + k * MC, 8)
        pltpu.sync_copy(bs, out_s.at[pl.ds(off, MC)])
        pltpu.sync_copy(bd, out_d.at[pl.ds(off, MC)])

    fire(0, bsa, bda)

    def body(jj, _):
        k = 2 * jj
        fire(k + 1, bsb, bdb)
        drain(k, bsa, bda)
        fire(k + 2, bsa, bda)
        drain(k + 1, bsb, bdb)
        return 0

    lax.fori_loop(0, (JG - 1) // 2, body, 0)
    drain(JG - 1, bsa, bda)


def _sc_gather_call(table, src3d, dst3d):
    return pl.kernel(
        _sc_gather_body,
        out_type=(
            jax.ShapeDtypeStruct((NEDGES, 16), F32),
            jax.ShapeDtypeStruct((NEDGES, 16), F32),
        ),
        mesh=plsc.VectorSubcoreMesh(
            core_axis_name="c", subcore_axis_name="s",
            num_cores=NC, num_subcores=NS),
        scratch_types=[
            pltpu.VMEM((JCH, CHUNK), I32),
            pltpu.VMEM((JCH, CHUNK), I32),
            pltpu.VMEM((MC, 16), F32),
            pltpu.VMEM((MC, 16), F32),
            pltpu.VMEM((MC, 16), F32),
            pltpu.VMEM((MC, 16), F32),
            pltpu.SemaphoreType.DMA,
            pltpu.SemaphoreType.DMA,
        ],
        compiler_params=pltpu.CompilerParams(use_tc_tiling_on_sc=False),
    )(table, src3d, dst3d)


_BE = 4000
_C1 = math.sqrt(3.0)
_C15 = math.sqrt(15.0)
_C5 = math.sqrt(5.0)
_ALPHA = 1.0 / math.sqrt(U * U)


def _dense_body(src, dst, shifts, cell_h, cell_l,
                w1, b1, w2, b2, w3, b3, w4p, b4p,
                r1m, r2m, mm, rgm, rsm, ram, rbm,
                w13m, w23m, b72m, sc72, out):
    pos_s = src[:, 0:3]
    ai_s = src[:, 3:7]
    bf = src[:, 7:8]
    pos_d = dst[:, 0:3]
    ai_d = dst[:, 3:7]

    gids = lax.broadcasted_iota(I32, (_BE, NGRAPH), 1).astype(F32)
    onehot_g = (bf == gids).astype(F32)
    oh_exp = _dot(onehot_g, rgm[...], _PD)
    sh_h, sh_l = _split(shifts[...])
    se_h = oh_exp * _dot(sh_h, rsm[...], _PD)
    se_l = oh_exp * _dot(sh_l, rsm[...], _PD)
    sv = (_dot(se_h, cell_h[...], _PD) + _dot(se_h, cell_l[...], _PD)
          + _dot(se_l, cell_h[...], _PD))
    vec = pos_d - pos_s + sv
    r2 = jnp.sum(vec * vec, axis=1, keepdims=True)
    r = jnp.sqrt(r2)
    u3 = vec * (1.0 / jnp.maximum(r, 1e-12))
    u3h, u3l = _split(u3)
    uu = ((_dot(u3h, ram[...], _PD) + _dot(u3l, ram[...], _PD))
          * (_dot(u3h, rbm[...], _PD) + _dot(u3l, rbm[...], _PD)))
    sh_exp = (b72m[...]
              + _dot(u3h, w13m[...], _PD) + _dot(u3l, w13m[...], _PD)
              + _dx(uu, w23m[...]))

    xr = jnp.minimum(r * (1.0 / RMAX), 1.0)
    centers = (lax.broadcasted_iota(I32, (_BE, NB), 1).astype(F32)
               / float(NB - 1))
    dx = (xr - centers) * float(NB - 1)
    emb = jnp.exp(-0.5 * dx * dx)
    emb = emb * jnp.where(r <= RMAX, float(NB ** 0.5), 0.0)

    g = _dot(emb, w1[...], _PD) + b1[...]
    g = g / (1.0 + jnp.exp(-g))
    g = _dot(g, w2[...], _PD) + b2[...]
    g = g / (1.0 + jnp.exp(-g))
    g = _dot(g, w3[...], _PD) + b3[...]
    g = g / (1.0 + jnp.exp(-g))
    w2e = _dot(g, w4p[...], _PD) + b4p[...]

    p_exp = _dx(ai_s, r1m[...]) * _dx(ai_d, r2m[...])
    t = p_exp * w2e
    s72 = _dx(t, mm[...])
    feat = sh_exp * s72 * sc72[...]
    out[:, 0:FEATW] = feat
    out[:, FEATW:FEATW + 1] = jnp.ones((_BE, 1), F32)
    out[:, FEATW + 1:FPAD] = jnp.zeros((_BE, FPAD - FEATW - 1), F32)


def _dense_call(src_rows, dst_rows, shifts, cell_cm,
                w1, b1, w2, b2, w3, b3, w4p, b4p):
    grid = (NEDGES // _BE,)
    full = lambda a: pl.BlockSpec(a.shape, lambda i: tuple(0 for _ in a.shape))
    cell_h = cell_cm.astype(jnp.bfloat16).astype(F32)
    cell_l = cell_cm - cell_h
    consts = [jnp.asarray(x) for x in
              (_R1, _R2, _M, _RG, _RS, _RA, _RB,
               _W13, _W23, _B72, _SCALE72)]
    return pl.pallas_call(
        _dense_body,
        grid=grid,
        in_specs=[
            pl.BlockSpec((_BE, 16), lambda i: (i, 0)),
            pl.BlockSpec((_BE, 16), lambda i: (i, 0)),
            pl.BlockSpec((_BE, 3), lambda i: (i, 0)),
            full(cell_h), full(cell_l), full(w1), full(b1), full(w2),
            full(b2), full(w3), full(b3), full(w4p), full(b4p),
        ] + [full(x) for x in consts],
        out_specs=pl.BlockSpec((_BE, FPAD), lambda i: (i, 0)),
        out_shape=jax.ShapeDtypeStruct((NEDGES, FPAD), F32),
    )(src_rows, dst_rows, shifts, cell_h, cell_l, w1, b1, w2, b2, w3, b3,
      w4p, b4p, *consts)


_ZR = 128


def _sc_scatter_body(feat, dst3d, out, acc, idx, fba, fbb, zbuf, sem):
    c = lax.axis_index("c")
    s = lax.axis_index("s")
    wid = s * NC + c

    def zrow(i, _):
        for k in range(FPAD // 16):
            zbuf[i, pl.ds(k * 16, 16)] = jnp.zeros((16,), F32)
        return 0

    lax.fori_loop(0, _ZR, zrow, 0)
    srow = pl.multiple_of(s * NROWS_PT, 8)

    def zcp(q, _):
        pltpu.sync_copy(zbuf, acc.at[pl.ds(srow + q * _ZR, _ZR)])
        return 0

    lax.fori_loop(0, NROWS_PT // _ZR, zcp, 0)
    plsc.subcore_barrier()

    pltpu.sync_copy(dst3d.at[wid], idx)
    base = pl.multiple_of(wid * EPW, 8)

    def fire(j, fb):
        off = pl.multiple_of(base + j * PAIR, 8)
        pltpu.async_copy(feat.at[pl.ds(off, PAIR)], fb, sem)

    def drain(j, fb):
        off = pl.multiple_of(base + j * PAIR, 8)
        pltpu.make_async_copy(feat.at[pl.ds(off, PAIR)], fb, sem).wait()
        pltpu.sync_copy(fb.at[pl.ds(0, CHUNK)],
                        acc.at[idx.at[2 * j]], add=True)
        pltpu.sync_copy(fb.at[pl.ds(CHUNK, CHUNK)],
                        acc.at[idx.at[2 * j + 1]], add=True)

    fire(0, fba)

    def body(jj, _):
        j = 2 * jj
        fire(j + 1, fbb)
        drain(j, fba)
        fire(j + 2, fba)
        drain(j + 1, fbb)
        return 0

    lax.fori_loop(0, (JP - 1) // 2, body, 0)
    drain(JP - 1, fba)
    plsc.subcore_barrier()
    pltpu.sync_copy(acc.at[pl.ds(srow, NROWS_PT)],
                    out.at[c, pl.ds(srow, NROWS_PT)])


def _sc_scatter_call(feat, dst3d):
    return pl.kernel(
        _sc_scatter_body,
        out_type=jax.ShapeDtypeStruct((NC, NPAD, FPAD), F32),
        mesh=plsc.VectorSubcoreMesh(
            core_axis_name="c", subcore_axis_name="s",
            num_cores=NC, num_subcores=NS),
        scratch_types=[
            pltpu.VMEM_SHARED((NPAD, FPAD), F32),
            pltpu.VMEM((JCH, CHUNK), I32),
            pltpu.VMEM((PAIR, FPAD), F32),
            pltpu.VMEM((PAIR, FPAD), F32),
            pltpu.VMEM((_ZR, FPAD), F32),
            pltpu.SemaphoreType.DMA,
        ],
        compiler_params=pltpu.CompilerParams(use_tc_tiling_on_sc=False),
    )(feat, dst3d)


def _final_body(parts, out):
    tot = parts[0, 0:NNODES, :] + parts[1, 0:NNODES, :]
    cnt = jnp.maximum(tot[:, FEATW:FEATW + 1], 1.0)
    out[...] = tot[:, 0:FEATW] / cnt


def _final_call(parts):
    return pl.pallas_call(
        _final_body,
        out_shape=jax.ShapeDtypeStruct((NNODES, FEATW), F32),
    )(parts)


def kernel(pos, A, batch, edge_src, edge_dst, edge_shifts, cell, emb_table,
           fit_W1, fit_b1, fit_W2, fit_b2, fit_W3, fit_b3,
           fc_W1, fc_b1, fc_W2, fc_b2, fc_W3, fc_b3, fc_W4, fc_b4):
    a2d = A.reshape(NNODES, 1)
    bf2d = batch.astype(F32).reshape(NNODES, 1)
    emb_pad = jnp.zeros((16, EMBD), F32).at[:MAXA].set(emb_table)
    node_table = _prep_call(
        pos, a2d, bf2d, emb_pad,
        fit_W1, fit_b1.reshape(1, -1), fit_W2, fit_b2.reshape(1, -1),
        fit_W3, fit_b3.reshape(1, -1))

    src3d = edge_src.reshape(NW, JCH, CHUNK)
    dst3d = edge_dst.reshape(NW, JCH, CHUNK)
    src_rows, dst_rows = _sc_gather_call(node_table, src3d, dst3d)

    w4p = fc_W4.reshape(64, 3, U, U, OUTM).transpose(0, 2, 3, 1, 4)
    w4p = w4p.reshape(64, 3 * U * U * OUTM)
    b4p = fc_b4.reshape(3, U, U, OUTM).transpose(1, 2, 0, 3).reshape(1, -1)
    cell_cm = cell.reshape(NGRAPH * 3, 3)
    feat = _dense_call(
        src_rows, dst_rows, edge_shifts, cell_cm,
        fc_W1, fc_b1.reshape(1, -1), fc_W2, fc_b2.reshape(1, -1),
        fc_W3, fc_b3.reshape(1, -1), w4p, b4p)

    parts = _sc_scatter_call(feat, dst3d)
    return _final_call(parts)
